# baseline (speedup 1.0000x reference)
import jax
import jax.numpy as jnp
from jax import lax
from jax.experimental import pallas as pl
from jax.experimental.pallas import tpu as pltpu
from jax.experimental.pallas import tpu_sc as plsc

f32 = jnp.float32
i32 = jnp.int32

NC = 2
NS = 16
NW = NC * NS

NP = 10240
B = 192
NB = 28
EPW = NB * B
EP = NW * EPW

D = 640
CH = 80
NCH = D // CH
SLICE = NP // NS


_SC_PARAMS = pltpu.CompilerParams(use_tc_tiling_on_sc=False,
                                  needs_layout_passes=False)


def _splat(val, n=16, dtype=i32):
    return jnp.full((n,), val, dtype=dtype)


def _sc_deg(dstb, ewf, z16, degp, dst2_v, ew_v, rows_v, deg_s):
    c = lax.axis_index("c")
    s = lax.axis_index("s")
    w = c * NS + s
    sl = pl.ds(s * SLICE, SLICE)
    pltpu.sync_copy(z16.at[sl], deg_s.at[sl])
    pltpu.sync_copy(dstb.at[w], dst2_v)
    pltpu.sync_copy(ewf.at[pl.ds(w * EPW, EPW)], ew_v)
    plsc.subcore_barrier()
    for j in range(NB):
        def ebody(e, _, j=j):
            ev = plsc.load_gather(ew_v, [_splat(e) + (j * B)])
            rows_v[e, :] = ev
            return 0
        lax.fori_loop(0, B, ebody, 0)
        pltpu.sync_copy(rows_v, deg_s.at[dst2_v.at[j]], add=True)
    plsc.subcore_barrier()
    pltpu.sync_copy(deg_s.at[sl], degp.at[c, sl])


def _tc_dinv(degp_ref, dinv_ref):
    deg = degp_ref[0][:, 0] + degp_ref[1][:, 0]
    dinv_ref[:] = jnp.where(deg > 0, lax.rsqrt(deg), 0.0)


def _compute_norm(src_v, dst2_v, ew_v, dinv_v, norm_v):
    def jbody(j, _):
        def nbody(i, _):
            ds16 = pl.ds(i * 16, 16)
            fl = pl.ds(j * B + i * 16, 16)
            sv = src_v[fl]
            dv = dst2_v[j, ds16]
            norm_v[fl] = (plsc.load_gather(dinv_v, [sv]) * ew_v[fl] *
                          plsc.load_gather(dinv_v, [dv]))
            return 0
        return lax.fori_loop(0, B // 16, nbody, 0)
    lax.fori_loop(0, NB, jbody, 0)


def _sc_norm_aggx(srcf, dstb, ewf, dinv, xpad, z16, normf, aggxp,
                  src_v, dst2_v, ew_v, norm_v, dinv_v, rows_v, agg_s,
                  sem):
    c = lax.axis_index("c")
    s = lax.axis_index("s")
    w = c * NS + s
    sl = pl.ds(s * SLICE, SLICE)
    pltpu.sync_copy(z16.at[sl], agg_s.at[sl])
    pltpu.sync_copy(srcf.at[pl.ds(w * EPW, EPW)], src_v)
    pltpu.sync_copy(dstb.at[w], dst2_v)
    pltpu.sync_copy(ewf.at[pl.ds(w * EPW, EPW)], ew_v)
    pltpu.sync_copy(dinv, dinv_v)
    plsc.subcore_barrier()
    _compute_norm(src_v, dst2_v, ew_v, dinv_v, norm_v)
    pltpu.sync_copy(norm_v, normf.at[pl.ds(w * EPW, EPW)])

    def jbody(j, _):
        pltpu.async_copy(xpad.at[src_v.at[pl.ds(j * B, B)]], rows_v,
                         sem).wait()

        def sbody(e, _):
            nsp = plsc.load_gather(norm_v, [_splat(e) + j * B])
            rows_v[e, :] = rows_v[e, :] * nsp
            return 0
        lax.fori_loop(0, B, sbody, 0)
        pltpu.sync_copy(rows_v, agg_s.at[dst2_v.at[j]], add=True)
        return 0
    lax.fori_loop(0, NB, jbody, 0)

    plsc.subcore_barrier()
    pltpu.sync_copy(agg_s.at[sl], aggxp.at[c, sl])


def _tc_layer1(aggxp_ref, w1_ref, b1_ref, out_ref):
    a = aggxp_ref[0] + aggxp_ref[1]
    h = jnp.dot(a, w1_ref[:], preferred_element_type=f32) + b1_ref[:][None, :]
    h = jnp.maximum(h, 0.0)
    for cidx in range(NCH):
        out_ref[cidx] = h[:, cidx * CH:(cidx + 1) * CH]


def _sc_agg2(srcf, dstb, normf, ident4, z80, tp,
             src_v, dst2_v, norm_v, rows_a, rows_b,
             acc_s, sem_a, sem_b, sem_s):
    c = lax.axis_index("c")
    s = lax.axis_index("s")
    w = c * NS + s
    sl = pl.ds(s * SLICE, SLICE)
    pltpu.sync_copy(srcf.at[pl.ds(w * EPW, EPW)], src_v)
    pltpu.sync_copy(dstb.at[w], dst2_v)
    pltpu.sync_copy(normf.at[pl.ds(w * EPW, EPW)], norm_v)

    def scale_batch(buf, j):
        def sbody(e, _):
            nsp = plsc.load_gather(norm_v, [_splat(e) + j * B])
            for k in range(CH // 16):
                ds16 = pl.ds(k * 16, 16)
                buf[e, ds16] = buf[e, ds16] * nsp
            return 0
        lax.fori_loop(0, B, sbody, 0, unroll=4)

    for cidx in range(NCH):
        pltpu.sync_copy(z80.at[sl], acc_s.at[sl])
        plsc.subcore_barrier()

        def gather(j, buf, sem, cidx=cidx):
            return pltpu.make_async_copy(
                ident4.at[cidx].at[src_v.at[pl.ds(j * B, B)]], buf, sem)

        gather(0, rows_a, sem_a).start()

        def scat(j, buf):
            return pltpu.make_async_copy(buf, acc_s.at[dst2_v.at[j]], sem_s)

        def pair(jj, _):
            j0 = 2 * jj
            j1 = 2 * jj + 1
            gather(j1, rows_b, sem_b).start()
            gather(j0, rows_a, sem_a).wait()
            scale_batch(rows_a, j0)
            pltpu.async_copy(rows_a, acc_s.at[dst2_v.at[j0]], sem_s,
                             add=True)
            gather(j1, rows_b, sem_b).wait()
            scale_batch(rows_b, j1)
            scat(j0, rows_a).wait()
            gather(jnp.minimum(j0 + 2, NB - 1), rows_a, sem_a).start()
            pltpu.sync_copy(rows_b, acc_s.at[dst2_v.at[j1]], add=True)
            return 0
        lax.fori_loop(0, NB // 2, pair, 0)
        gather(NB - 1, rows_a, sem_a).wait()
        plsc.subcore_barrier()
        pltpu.sync_copy(acc_s.at[sl], tp.at[c, cidx, sl])
        plsc.subcore_barrier()


def _tc_layer2(tp_ref, w2_ref, b2_ref, ident4_ref, we_ref, be_ref, out_ref):
    rows = tp_ref.shape[2]
    acc = jnp.zeros((rows, D), f32)
    for p in range(NC):
        for cidx in range(NCH):
            acc = acc + jnp.dot(tp_ref[p, cidx],
                                w2_ref[cidx * CH:(cidx + 1) * CH, :],
                                preferred_element_type=f32)
    h2 = jnp.maximum(acc + b2_ref[:][None, :], 0.0)
    o = jnp.dot(h2, we_ref[:D, :], preferred_element_type=f32)
    for cidx in range(NCH):
        o = o + jnp.dot(ident4_ref[cidx],
                        we_ref[D + cidx * CH:D + (cidx + 1) * CH, :],
                        preferred_element_type=f32)
    out_ref[:] = o + be_ref[:][None, :]


def kernel(x, edge_index, edge_weight, W1, b1, W2, b2, We, be):
    N, F = x.shape
    E = edge_weight.shape[0]

    src = edge_index[0].astype(i32)
    dst = edge_index[1].astype(i32)
    loop_idx = jnp.arange(N, dtype=i32)
    npad = EP - (E + N)
    pad_idx = (jnp.arange(npad, dtype=i32) * 37) % N
    srcf = jnp.concatenate([src, loop_idx, pad_idx])
    dstb = jnp.concatenate([dst, loop_idx, pad_idx]).reshape(NW, NB, B)
    ewf = jnp.concatenate([edge_weight.astype(f32), jnp.ones((N,), f32),
                           jnp.zeros((npad,), f32)])
    xpad = jnp.zeros((NP, 16), f32).at[:N, :F].set(x)
    w1p = jnp.zeros((16, D), f32).at[:F, :].set(W1)
    z16 = jnp.zeros((NP, 16), f32)
    z80 = jnp.zeros((NP, CH), f32)

    mesh = plsc.VectorSubcoreMesh(core_axis_name="c", subcore_axis_name="s")

    degp = pl.kernel(
        _sc_deg,
        out_type=jax.ShapeDtypeStruct((NC, NP, 16), f32),
        mesh=mesh,
        compiler_params=_SC_PARAMS,
        scratch_types=[
            pltpu.VMEM((NB, B), i32),
            pltpu.VMEM((EPW,), f32),
            pltpu.VMEM((B, 16), f32),
            pltpu.VMEM_SHARED((NP, 16), f32),
        ],
    )(dstb, ewf, z16)

    dinv = pl.pallas_call(
        _tc_dinv,
        out_shape=jax.ShapeDtypeStruct((NP,), f32),
    )(degp)

    normf, aggxp = pl.kernel(
        _sc_norm_aggx,
        out_type=(jax.ShapeDtypeStruct((EP,), f32),
                  jax.ShapeDtypeStruct((NC, NP, 16), f32)),
        mesh=mesh,
        compiler_params=_SC_PARAMS,
        scratch_types=[
            pltpu.VMEM((EPW,), i32),
            pltpu.VMEM((NB, B), i32),
            pltpu.VMEM((EPW,), f32),
            pltpu.VMEM((EPW,), f32),
            pltpu.VMEM((NP,), f32),
            pltpu.VMEM((B, 16), f32),
            pltpu.VMEM_SHARED((NP, 16), f32),
            pltpu.SemaphoreType.DMA,
        ],
    )(srcf, dstb, ewf, dinv, xpad, z16)

    R = 512
    ident4 = pl.pallas_call(
        _tc_layer1,
        grid=(NP // R,),
        in_specs=[
            pl.BlockSpec((NC, R, 16), lambda i: (0, i, 0)),
            pl.BlockSpec((16, D), lambda i: (0, 0)),
            pl.BlockSpec((D,), lambda i: (0,)),
        ],
        out_specs=pl.BlockSpec((NCH, R, CH), lambda i: (0, i, 0)),
        out_shape=jax.ShapeDtypeStruct((NCH, NP, CH), f32),
    )(aggxp, w1p, b1)

    tp = pl.kernel(
        _sc_agg2,
        out_type=jax.ShapeDtypeStruct((NC, NCH, NP, CH), f32),
        mesh=mesh,
        compiler_params=_SC_PARAMS,
        scratch_types=[
            pltpu.VMEM((EPW,), i32),
            pltpu.VMEM((NB, B), i32),
            pltpu.VMEM((EPW,), f32),
            pltpu.VMEM((B, CH), f32),
            pltpu.VMEM((B, CH), f32),
            pltpu.VMEM_SHARED((NP, CH), f32),
            pltpu.SemaphoreType.DMA,
            pltpu.SemaphoreType.DMA,
            pltpu.SemaphoreType.DMA,
        ],
    )(srcf, dstb, normf, ident4, z80)

    outp = pl.pallas_call(
        _tc_layer2,
        grid=(NP // R,),
        in_specs=[
            pl.BlockSpec((NC, NCH, R, CH), lambda i: (0, 0, i, 0)),
            pl.BlockSpec((D, D), lambda i: (0, 0)),
            pl.BlockSpec((D,), lambda i: (0,)),
            pl.BlockSpec((NCH, R, CH), lambda i: (0, i, 0)),
            pl.BlockSpec((2 * D, 80), lambda i: (0, 0)),
            pl.BlockSpec((80,), lambda i: (0,)),
        ],
        out_specs=pl.BlockSpec((R, 80), lambda i: (i, 0)),
        out_shape=jax.ShapeDtypeStruct((NP, 80), f32),
    )(tp, W2, b2, ident4, We, be)

    return outp[:N]


# speedup vs baseline: 1.0056x; 1.0056x over previous
import jax
import jax.numpy as jnp
from jax import lax
from jax.experimental import pallas as pl
from jax.experimental.pallas import tpu as pltpu
from jax.experimental.pallas import tpu_sc as plsc

f32 = jnp.float32
i32 = jnp.int32

NC = 2
NS = 16
NW = NC * NS

NP = 10240
B = 192
NB = 28
EPW = NB * B
EP = NW * EPW

D = 640
CH = 80
NCH = D // CH
SLICE = NP // NS


_SC_PARAMS = pltpu.CompilerParams(use_tc_tiling_on_sc=False,
                                  needs_layout_passes=False)


def _splat(val, n=16, dtype=i32):
    return jnp.full((n,), val, dtype=dtype)


def _sc_deg(dstb, ewf, z16, degp, dst2_v, ew_v, rows_v, deg_s):
    c = lax.axis_index("c")
    s = lax.axis_index("s")
    w = c * NS + s
    sl = pl.ds(s * SLICE, SLICE)
    pltpu.sync_copy(z16.at[sl], deg_s.at[sl])
    pltpu.sync_copy(dstb.at[w], dst2_v)
    pltpu.sync_copy(ewf.at[pl.ds(w * EPW, EPW)], ew_v)
    plsc.subcore_barrier()
    for j in range(NB):
        def ebody(e, _, j=j):
            ev = plsc.load_gather(ew_v, [_splat(e) + (j * B)])
            rows_v[e, :] = ev
            return 0
        lax.fori_loop(0, B, ebody, 0)
        pltpu.sync_copy(rows_v, deg_s.at[dst2_v.at[j]], add=True)
    plsc.subcore_barrier()
    pltpu.sync_copy(deg_s.at[sl], degp.at[c, sl])


def _tc_dinv(degp_ref, dinv_ref):
    deg = degp_ref[0][:, 0] + degp_ref[1][:, 0]
    dinv_ref[:] = jnp.where(deg > 0, lax.rsqrt(deg), 0.0)


def _compute_norm(src_v, dst2_v, ew_v, dinv_v, norm_v):
    def jbody(j, _):
        def nbody(i, _):
            ds16 = pl.ds(i * 16, 16)
            fl = pl.ds(j * B + i * 16, 16)
            sv = src_v[fl]
            dv = dst2_v[j, ds16]
            norm_v[fl] = (plsc.load_gather(dinv_v, [sv]) * ew_v[fl] *
                          plsc.load_gather(dinv_v, [dv]))
            return 0
        return lax.fori_loop(0, B // 16, nbody, 0)
    lax.fori_loop(0, NB, jbody, 0)


def _sc_norm_aggx(srcf, dstb, ewf, dinv, xpad, z16, normf, aggxp,
                  src_v, dst2_v, ew_v, norm_v, dinv_v, rows_v, agg_s,
                  sem):
    c = lax.axis_index("c")
    s = lax.axis_index("s")
    w = c * NS + s
    sl = pl.ds(s * SLICE, SLICE)
    pltpu.sync_copy(z16.at[sl], agg_s.at[sl])
    pltpu.sync_copy(srcf.at[pl.ds(w * EPW, EPW)], src_v)
    pltpu.sync_copy(dstb.at[w], dst2_v)
    pltpu.sync_copy(ewf.at[pl.ds(w * EPW, EPW)], ew_v)
    pltpu.sync_copy(dinv, dinv_v)
    plsc.subcore_barrier()
    _compute_norm(src_v, dst2_v, ew_v, dinv_v, norm_v)
    pltpu.sync_copy(norm_v, normf.at[pl.ds(w * EPW, EPW)])

    def jbody(j, _):
        pltpu.async_copy(xpad.at[src_v.at[pl.ds(j * B, B)]], rows_v,
                         sem).wait()

        def sbody(e, _):
            nsp = plsc.load_gather(norm_v, [_splat(e) + j * B])
            rows_v[e, :] = rows_v[e, :] * nsp
            return 0
        lax.fori_loop(0, B, sbody, 0)
        pltpu.sync_copy(rows_v, agg_s.at[dst2_v.at[j]], add=True)
        return 0
    lax.fori_loop(0, NB, jbody, 0)

    plsc.subcore_barrier()
    pltpu.sync_copy(agg_s.at[sl], aggxp.at[c, sl])


def _tc_layer1(aggxp_ref, w1_ref, b1_ref, out_ref):
    a = aggxp_ref[0] + aggxp_ref[1]
    h = jnp.dot(a, w1_ref[:], preferred_element_type=f32) + b1_ref[:][None, :]
    h = jnp.maximum(h, 0.0)
    for cidx in range(NCH):
        out_ref[cidx] = h[:, cidx * CH:(cidx + 1) * CH]


def _sc_agg2(srcf, dstb, normf, ident4, z80, tp,
             src_v, dst2_v, norm_v, rows_a, rows_b,
             acc_s, sem_a, sem_b, sem_s):
    c = lax.axis_index("c")
    s = lax.axis_index("s")
    w = c * NS + s
    sl = pl.ds(s * SLICE, SLICE)
    pltpu.sync_copy(srcf.at[pl.ds(w * EPW, EPW)], src_v)
    pltpu.sync_copy(dstb.at[w], dst2_v)
    pltpu.sync_copy(normf.at[pl.ds(w * EPW, EPW)], norm_v)

    def scale_batch(buf, j):
        def sbody(e, _):
            nsp = plsc.load_gather(norm_v, [_splat(e) + j * B])
            for k in range(CH // 16):
                ds16 = pl.ds(k * 16, 16)
                buf[e, ds16] = buf[e, ds16] * nsp
            return 0
        lax.fori_loop(0, B, sbody, 0, unroll=2)

    for cidx in range(NCH):
        pltpu.sync_copy(z80.at[sl], acc_s.at[sl])
        plsc.subcore_barrier()

        def gather(j, buf, sem, cidx=cidx):
            return pltpu.make_async_copy(
                ident4.at[cidx].at[src_v.at[pl.ds(j * B, B)]], buf, sem)

        gather(0, rows_a, sem_a).start()

        def scat(j, buf):
            return pltpu.make_async_copy(buf, acc_s.at[dst2_v.at[j]], sem_s)

        def pair(jj, _):
            j0 = 2 * jj
            j1 = 2 * jj + 1
            gather(j1, rows_b, sem_b).start()
            gather(j0, rows_a, sem_a).wait()
            scale_batch(rows_a, j0)
            pltpu.async_copy(rows_a, acc_s.at[dst2_v.at[j0]], sem_s,
                             add=True)
            gather(j1, rows_b, sem_b).wait()
            scale_batch(rows_b, j1)
            scat(j0, rows_a).wait()
            gather(jnp.minimum(j0 + 2, NB - 1), rows_a, sem_a).start()
            pltpu.sync_copy(rows_b, acc_s.at[dst2_v.at[j1]], add=True)
            return 0
        lax.fori_loop(0, NB // 2, pair, 0)
        gather(NB - 1, rows_a, sem_a).wait()
        plsc.subcore_barrier()
        pltpu.sync_copy(acc_s.at[sl], tp.at[c, cidx, sl])
        plsc.subcore_barrier()


def _tc_layer2(tp_ref, w2_ref, b2_ref, ident4_ref, we_ref, be_ref, out_ref):
    rows = tp_ref.shape[2]
    acc = jnp.zeros((rows, D), f32)
    for p in range(NC):
        for cidx in range(NCH):
            acc = acc + jnp.dot(tp_ref[p, cidx],
                                w2_ref[cidx * CH:(cidx + 1) * CH, :],
                                preferred_element_type=f32)
    h2 = jnp.maximum(acc + b2_ref[:][None, :], 0.0)
    o = jnp.dot(h2, we_ref[:D, :], preferred_element_type=f32)
    for cidx in range(NCH):
        o = o + jnp.dot(ident4_ref[cidx],
                        we_ref[D + cidx * CH:D + (cidx + 1) * CH, :],
                        preferred_element_type=f32)
    out_ref[:] = o + be_ref[:][None, :]


def kernel(x, edge_index, edge_weight, W1, b1, W2, b2, We, be):
    N, F = x.shape
    E = edge_weight.shape[0]

    src = edge_index[0].astype(i32)
    dst = edge_index[1].astype(i32)
    loop_idx = jnp.arange(N, dtype=i32)
    npad = EP - (E + N)
    pad_idx = (jnp.arange(npad, dtype=i32) * 37) % N
    srcf = jnp.concatenate([src, loop_idx, pad_idx])
    dstb = jnp.concatenate([dst, loop_idx, pad_idx]).reshape(NW, NB, B)
    ewf = jnp.concatenate([edge_weight.astype(f32), jnp.ones((N,), f32),
                           jnp.zeros((npad,), f32)])
    xpad = jnp.zeros((NP, 16), f32).at[:N, :F].set(x)
    w1p = jnp.zeros((16, D), f32).at[:F, :].set(W1)
    z16 = jnp.zeros((NP, 16), f32)
    z80 = jnp.zeros((NP, CH), f32)

    mesh = plsc.VectorSubcoreMesh(core_axis_name="c", subcore_axis_name="s")

    degp = pl.kernel(
        _sc_deg,
        out_type=jax.ShapeDtypeStruct((NC, NP, 16), f32),
        mesh=mesh,
        compiler_params=_SC_PARAMS,
        scratch_types=[
            pltpu.VMEM((NB, B), i32),
            pltpu.VMEM((EPW,), f32),
            pltpu.VMEM((B, 16), f32),
            pltpu.VMEM_SHARED((NP, 16), f32),
        ],
    )(dstb, ewf, z16)

    dinv = pl.pallas_call(
        _tc_dinv,
        out_shape=jax.ShapeDtypeStruct((NP,), f32),
    )(degp)

    normf, aggxp = pl.kernel(
        _sc_norm_aggx,
        out_type=(jax.ShapeDtypeStruct((EP,), f32),
                  jax.ShapeDtypeStruct((NC, NP, 16), f32)),
        mesh=mesh,
        compiler_params=_SC_PARAMS,
        scratch_types=[
            pltpu.VMEM((EPW,), i32),
            pltpu.VMEM((NB, B), i32),
            pltpu.VMEM((EPW,), f32),
            pltpu.VMEM((EPW,), f32),
            pltpu.VMEM((NP,), f32),
            pltpu.VMEM((B, 16), f32),
            pltpu.VMEM_SHARED((NP, 16), f32),
            pltpu.SemaphoreType.DMA,
        ],
    )(srcf, dstb, ewf, dinv, xpad, z16)

    R = 512
    ident4 = pl.pallas_call(
        _tc_layer1,
        grid=(NP // R,),
        in_specs=[
            pl.BlockSpec((NC, R, 16), lambda i: (0, i, 0)),
            pl.BlockSpec((16, D), lambda i: (0, 0)),
            pl.BlockSpec((D,), lambda i: (0,)),
        ],
        out_specs=pl.BlockSpec((NCH, R, CH), lambda i: (0, i, 0)),
        out_shape=jax.ShapeDtypeStruct((NCH, NP, CH), f32),
    )(aggxp, w1p, b1)

    tp = pl.kernel(
        _sc_agg2,
        out_type=jax.ShapeDtypeStruct((NC, NCH, NP, CH), f32),
        mesh=mesh,
        compiler_params=_SC_PARAMS,
        scratch_types=[
            pltpu.VMEM((EPW,), i32),
            pltpu.VMEM((NB, B), i32),
            pltpu.VMEM((EPW,), f32),
            pltpu.VMEM((B, CH), f32),
            pltpu.VMEM((B, CH), f32),
            pltpu.VMEM_SHARED((NP, CH), f32),
            pltpu.SemaphoreType.DMA,
            pltpu.SemaphoreType.DMA,
            pltpu.SemaphoreType.DMA,
        ],
    )(srcf, dstb, normf, ident4, z80)

    outp = pl.pallas_call(
        _tc_layer2,
        grid=(NP // R,),
        in_specs=[
            pl.BlockSpec((NC, NCH, R, CH), lambda i: (0, 0, i, 0)),
            pl.BlockSpec((D, D), lambda i: (0, 0)),
            pl.BlockSpec((D,), lambda i: (0,)),
            pl.BlockSpec((NCH, R, CH), lambda i: (0, i, 0)),
            pl.BlockSpec((2 * D, 80), lambda i: (0, 0)),
            pl.BlockSpec((80,), lambda i: (0,)),
        ],
        out_specs=pl.BlockSpec((R, 80), lambda i: (i, 0)),
        out_shape=jax.ShapeDtypeStruct((NP, 80), f32),
    )(tp, W2, b2, ident4, We, be)

    return outp[:N]


# speedup vs baseline: 1.1262x; 1.1199x over previous
import jax
import jax.numpy as jnp
from jax import lax
from jax.experimental import pallas as pl
from jax.experimental.pallas import tpu as pltpu
from jax.experimental.pallas import tpu_sc as plsc

f32 = jnp.float32
i32 = jnp.int32

NC = 2
NS = 16
NW = NC * NS

NP = 10240
B = 192
NB = 30
EPW = NB * B
EP = NW * EPW

D = 640
CH = 80
NCH = D // CH
SLICE = NP // NS


_SC_PARAMS = pltpu.CompilerParams(use_tc_tiling_on_sc=False,
                                  needs_layout_passes=False)


def _splat(val, n=16, dtype=i32):
    return jnp.full((n,), val, dtype=dtype)


def _sc_deg(dstb, ewf, z16, degp, dst2_v, ew_v, rows_v, deg_s):
    c = lax.axis_index("c")
    s = lax.axis_index("s")
    w = c * NS + s
    sl = pl.ds(s * SLICE, SLICE)
    pltpu.sync_copy(z16.at[sl], deg_s.at[sl])
    pltpu.sync_copy(dstb.at[w], dst2_v)
    pltpu.sync_copy(ewf.at[pl.ds(w * EPW, EPW)], ew_v)
    plsc.subcore_barrier()
    for j in range(NB):
        def ebody(e, _, j=j):
            ev = plsc.load_gather(ew_v, [_splat(e) + (j * B)])
            rows_v[e, :] = ev
            return 0
        lax.fori_loop(0, B, ebody, 0)
        pltpu.sync_copy(rows_v, deg_s.at[dst2_v.at[j]], add=True)
    plsc.subcore_barrier()
    pltpu.sync_copy(deg_s.at[sl], degp.at[c, sl])


def _tc_dinv(degp_ref, dinv_ref):
    deg = degp_ref[0][:, 0] + degp_ref[1][:, 0]
    dinv_ref[:] = jnp.where(deg > 0, lax.rsqrt(deg), 0.0)


def _compute_norm(src_v, dst2_v, ew_v, dinv_v, norm_v):
    def jbody(j, _):
        def nbody(i, _):
            ds16 = pl.ds(i * 16, 16)
            fl = pl.ds(j * B + i * 16, 16)
            sv = src_v[fl]
            dv = dst2_v[j, ds16]
            norm_v[fl] = (plsc.load_gather(dinv_v, [sv]) * ew_v[fl] *
                          plsc.load_gather(dinv_v, [dv]))
            return 0
        return lax.fori_loop(0, B // 16, nbody, 0)
    lax.fori_loop(0, NB, jbody, 0)


def _sc_norm_aggx(srcf, dstb, ewf, dinv, xpad, z16, normf, aggxp,
                  src_v, dst2_v, ew_v, norm_v, dinv_v, rows_v, agg_s,
                  sem):
    c = lax.axis_index("c")
    s = lax.axis_index("s")
    w = c * NS + s
    sl = pl.ds(s * SLICE, SLICE)
    pltpu.sync_copy(z16.at[sl], agg_s.at[sl])
    pltpu.sync_copy(srcf.at[pl.ds(w * EPW, EPW)], src_v)
    pltpu.sync_copy(dstb.at[w], dst2_v)
    pltpu.sync_copy(ewf.at[pl.ds(w * EPW, EPW)], ew_v)
    pltpu.sync_copy(dinv, dinv_v)
    plsc.subcore_barrier()
    _compute_norm(src_v, dst2_v, ew_v, dinv_v, norm_v)
    pltpu.sync_copy(norm_v, normf.at[pl.ds(w * EPW, EPW)])

    def jbody(j, _):
        pltpu.async_copy(xpad.at[src_v.at[pl.ds(j * B, B)]], rows_v,
                         sem).wait()

        def sbody(e, _):
            nsp = plsc.load_gather(norm_v, [_splat(e) + j * B])
            rows_v[e, :] = rows_v[e, :] * nsp
            return 0
        lax.fori_loop(0, B, sbody, 0)
        pltpu.sync_copy(rows_v, agg_s.at[dst2_v.at[j]], add=True)
        return 0
    lax.fori_loop(0, NB, jbody, 0)

    plsc.subcore_barrier()
    pltpu.sync_copy(agg_s.at[sl], aggxp.at[c, sl])


def _tc_layer1(aggxp_ref, w1_ref, b1_ref, out_ref):
    a = aggxp_ref[0] + aggxp_ref[1]
    h = jnp.dot(a, w1_ref[:], preferred_element_type=f32) + b1_ref[:][None, :]
    h = jnp.maximum(h, 0.0)
    for cidx in range(NCH):
        out_ref[cidx] = h[:, cidx * CH:(cidx + 1) * CH]


def _sc_agg2(srcf, dstb, normf, ident4, z80, tp,
             src_v, dst2_v, norm_v, rows_a, rows_b, rows_c,
             acc_s, gsa, gsb, gsc, ssa, ssb, ssc):
    c = lax.axis_index("c")
    s = lax.axis_index("s")
    w = c * NS + s
    sl = pl.ds(s * SLICE, SLICE)
    pltpu.sync_copy(srcf.at[pl.ds(w * EPW, EPW)], src_v)
    pltpu.sync_copy(dstb.at[w], dst2_v)
    pltpu.sync_copy(normf.at[pl.ds(w * EPW, EPW)], norm_v)

    def scale_batch(buf, j):
        def sbody(e, _):
            nsp = plsc.load_gather(norm_v, [_splat(e) + j * B])
            for k in range(CH // 16):
                ds16 = pl.ds(k * 16, 16)
                buf[e, ds16] = buf[e, ds16] * nsp
            return 0
        lax.fori_loop(0, B, sbody, 0, unroll=2)

    for cidx in range(NCH):
        pltpu.sync_copy(z80.at[sl], acc_s.at[sl])
        plsc.subcore_barrier()

        def gather(j, buf, sem, cidx=cidx):
            jc = jnp.minimum(j, NB - 1)
            return pltpu.make_async_copy(
                ident4.at[cidx].at[src_v.at[pl.ds(jc * B, B)]], buf, sem)

        def scat(j, buf, sem):
            return pltpu.make_async_copy(buf, acc_s.at[dst2_v.at[j]], sem)

        gather(0, rows_a, gsa).start()
        gather(1, rows_b, gsb).start()

        def tri(jj, _):
            j0 = 3 * jj
            j1 = 3 * jj + 1
            j2 = 3 * jj + 2
            gather(j0, rows_a, gsa).wait()
            scale_batch(rows_a, j0)

            @pl.when(jj > 0)
            def _():
                scat(j2 - 3, rows_c, ssc).wait()
            gather(j2, rows_c, gsc).start()
            pltpu.async_copy(rows_a, acc_s.at[dst2_v.at[j0]], ssa, add=True)

            gather(j1, rows_b, gsb).wait()
            scale_batch(rows_b, j1)
            scat(j0, rows_a, ssa).wait()
            gather(j0 + 3, rows_a, gsa).start()
            pltpu.async_copy(rows_b, acc_s.at[dst2_v.at[j1]], ssb, add=True)

            gather(j2, rows_c, gsc).wait()
            scale_batch(rows_c, j2)
            scat(j1, rows_b, ssb).wait()
            gather(j1 + 3, rows_b, gsb).start()
            pltpu.async_copy(rows_c, acc_s.at[dst2_v.at[j2]], ssc, add=True)
            return 0
        lax.fori_loop(0, NB // 3, tri, 0)
        gather(NB - 1, rows_a, gsa).wait()
        gather(NB - 1, rows_b, gsb).wait()
        scat(NB - 1, rows_c, ssc).wait()
        plsc.subcore_barrier()
        pltpu.sync_copy(acc_s.at[sl], tp.at[c, cidx, sl])
        plsc.subcore_barrier()


def _tc_layer2(tp_ref, w2_ref, b2_ref, ident4_ref, we_ref, be_ref, out_ref):
    rows = tp_ref.shape[2]
    acc = jnp.zeros((rows, D), f32)
    for p in range(NC):
        for cidx in range(NCH):
            acc = acc + jnp.dot(tp_ref[p, cidx],
                                w2_ref[cidx * CH:(cidx + 1) * CH, :],
                                preferred_element_type=f32)
    h2 = jnp.maximum(acc + b2_ref[:][None, :], 0.0)
    o = jnp.dot(h2, we_ref[:D, :], preferred_element_type=f32)
    for cidx in range(NCH):
        o = o + jnp.dot(ident4_ref[cidx],
                        we_ref[D + cidx * CH:D + (cidx + 1) * CH, :],
                        preferred_element_type=f32)
    out_ref[:] = o + be_ref[:][None, :]


def kernel(x, edge_index, edge_weight, W1, b1, W2, b2, We, be):
    N, F = x.shape
    E = edge_weight.shape[0]

    src = edge_index[0].astype(i32)
    dst = edge_index[1].astype(i32)
    loop_idx = jnp.arange(N, dtype=i32)
    npad = EP - (E + N)
    pad_idx = (jnp.arange(npad, dtype=i32) * 37) % N
    srcf = jnp.concatenate([src, loop_idx, pad_idx])
    dstb = jnp.concatenate([dst, loop_idx, pad_idx]).reshape(NW, NB, B)
    ewf = jnp.concatenate([edge_weight.astype(f32), jnp.ones((N,), f32),
                           jnp.zeros((npad,), f32)])
    xpad = jnp.zeros((NP, 16), f32).at[:N, :F].set(x)
    w1p = jnp.zeros((16, D), f32).at[:F, :].set(W1)
    z16 = jnp.zeros((NP, 16), f32)
    z80 = jnp.zeros((NP, CH), f32)

    mesh = plsc.VectorSubcoreMesh(core_axis_name="c", subcore_axis_name="s")

    degp = pl.kernel(
        _sc_deg,
        out_type=jax.ShapeDtypeStruct((NC, NP, 16), f32),
        mesh=mesh,
        compiler_params=_SC_PARAMS,
        scratch_types=[
            pltpu.VMEM((NB, B), i32),
            pltpu.VMEM((EPW,), f32),
            pltpu.VMEM((B, 16), f32),
            pltpu.VMEM_SHARED((NP, 16), f32),
        ],
    )(dstb, ewf, z16)

    dinv = pl.pallas_call(
        _tc_dinv,
        out_shape=jax.ShapeDtypeStruct((NP,), f32),
    )(degp)

    normf, aggxp = pl.kernel(
        _sc_norm_aggx,
        out_type=(jax.ShapeDtypeStruct((EP,), f32),
                  jax.ShapeDtypeStruct((NC, NP, 16), f32)),
        mesh=mesh,
        compiler_params=_SC_PARAMS,
        scratch_types=[
            pltpu.VMEM((EPW,), i32),
            pltpu.VMEM((NB, B), i32),
            pltpu.VMEM((EPW,), f32),
            pltpu.VMEM((EPW,), f32),
            pltpu.VMEM((NP,), f32),
            pltpu.VMEM((B, 16), f32),
            pltpu.VMEM_SHARED((NP, 16), f32),
            pltpu.SemaphoreType.DMA,
        ],
    )(srcf, dstb, ewf, dinv, xpad, z16)

    R = 512
    ident4 = pl.pallas_call(
        _tc_layer1,
        grid=(NP // R,),
        in_specs=[
            pl.BlockSpec((NC, R, 16), lambda i: (0, i, 0)),
            pl.BlockSpec((16, D), lambda i: (0, 0)),
            pl.BlockSpec((D,), lambda i: (0,)),
        ],
        out_specs=pl.BlockSpec((NCH, R, CH), lambda i: (0, i, 0)),
        out_shape=jax.ShapeDtypeStruct((NCH, NP, CH), f32),
    )(aggxp, w1p, b1)

    tp = pl.kernel(
        _sc_agg2,
        out_type=jax.ShapeDtypeStruct((NC, NCH, NP, CH), f32),
        mesh=mesh,
        compiler_params=_SC_PARAMS,
        scratch_types=[
            pltpu.VMEM((EPW,), i32),
            pltpu.VMEM((NB, B), i32),
            pltpu.VMEM((EPW,), f32),
            pltpu.VMEM((B, CH), f32),
            pltpu.VMEM((B, CH), f32),
            pltpu.VMEM((B, CH), f32),
            pltpu.VMEM_SHARED((NP, CH), f32),
            pltpu.SemaphoreType.DMA,
            pltpu.SemaphoreType.DMA,
            pltpu.SemaphoreType.DMA,
            pltpu.SemaphoreType.DMA,
            pltpu.SemaphoreType.DMA,
            pltpu.SemaphoreType.DMA,
        ],
    )(srcf, dstb, normf, ident4, z80)

    outp = pl.pallas_call(
        _tc_layer2,
        grid=(NP // R,),
        in_specs=[
            pl.BlockSpec((NC, NCH, R, CH), lambda i: (0, 0, i, 0)),
            pl.BlockSpec((D, D), lambda i: (0, 0)),
            pl.BlockSpec((D,), lambda i: (0,)),
            pl.BlockSpec((NCH, R, CH), lambda i: (0, i, 0)),
            pl.BlockSpec((2 * D, 80), lambda i: (0, 0)),
            pl.BlockSpec((80,), lambda i: (0,)),
        ],
        out_specs=pl.BlockSpec((R, 80), lambda i: (i, 0)),
        out_shape=jax.ShapeDtypeStruct((NP, 80), f32),
    )(tp, W2, b2, ident4, We, be)

    return outp[:N]


# speedup vs baseline: 1.1637x; 1.0333x over previous
import jax
import jax.numpy as jnp
from jax import lax
from jax.experimental import pallas as pl
from jax.experimental.pallas import tpu as pltpu
from jax.experimental.pallas import tpu_sc as plsc

f32 = jnp.float32
i32 = jnp.int32

NC = 2
NS = 16
NW = NC * NS

NP = 10240
B = 192
NB = 30
EPW = NB * B
EP = NW * EPW

D = 640
CH = 80
NCH = D // CH
SLICE = NP // NS


_SC_PARAMS = pltpu.CompilerParams(use_tc_tiling_on_sc=False,
                                  needs_layout_passes=False)


def _splat(val, n=16, dtype=i32):
    return jnp.full((n,), val, dtype=dtype)


def _sc_deg(dstb, ewf, z16, degp, dst2_v, ew_v, rows_v, deg_s):
    c = lax.axis_index("c")
    s = lax.axis_index("s")
    w = c * NS + s
    sl = pl.ds(s * SLICE, SLICE)
    pltpu.sync_copy(z16.at[sl], deg_s.at[sl])
    pltpu.sync_copy(dstb.at[w], dst2_v)
    pltpu.sync_copy(ewf.at[pl.ds(w * EPW, EPW)], ew_v)
    plsc.subcore_barrier()
    for j in range(NB):
        def ebody(e, _, j=j):
            ev = plsc.load_gather(ew_v, [_splat(e) + (j * B)])
            rows_v[e, :] = ev
            return 0
        lax.fori_loop(0, B, ebody, 0)
        pltpu.sync_copy(rows_v, deg_s.at[dst2_v.at[j]], add=True)
    plsc.subcore_barrier()
    pltpu.sync_copy(deg_s.at[sl], degp.at[c, sl])


def _tc_dinv(degp_ref, dinv_ref):
    deg = degp_ref[0][:, 0] + degp_ref[1][:, 0]
    dinv_ref[:] = jnp.where(deg > 0, lax.rsqrt(deg), 0.0)


def _compute_norm(src_v, dst2_v, ew_v, dinv_v, norm_v):
    def jbody(j, _):
        def nbody(i, _):
            ds16 = pl.ds(i * 16, 16)
            fl = pl.ds(j * B + i * 16, 16)
            sv = src_v[fl]
            dv = dst2_v[j, ds16]
            norm_v[fl] = (plsc.load_gather(dinv_v, [sv]) * ew_v[fl] *
                          plsc.load_gather(dinv_v, [dv]))
            return 0
        return lax.fori_loop(0, B // 16, nbody, 0)
    lax.fori_loop(0, NB, jbody, 0)


def _sc_norm_aggx(srcf, dstb, ewf, dinv, xpad, z16, normf, aggxp,
                  src_v, dst2_v, ew_v, norm_v, dinv_v, rows_a, rows_b,
                  rows_c, agg_s, gsa, gsb, gsc, ssa, ssb, ssc):
    c = lax.axis_index("c")
    s = lax.axis_index("s")
    w = c * NS + s
    sl = pl.ds(s * SLICE, SLICE)
    pltpu.sync_copy(z16.at[sl], agg_s.at[sl])
    pltpu.sync_copy(srcf.at[pl.ds(w * EPW, EPW)], src_v)
    pltpu.sync_copy(dstb.at[w], dst2_v)
    pltpu.sync_copy(ewf.at[pl.ds(w * EPW, EPW)], ew_v)
    pltpu.sync_copy(dinv, dinv_v)
    plsc.subcore_barrier()
    _compute_norm(src_v, dst2_v, ew_v, dinv_v, norm_v)
    pltpu.sync_copy(norm_v, normf.at[pl.ds(w * EPW, EPW)])

    def gather(j, buf, gsem):
        jc = jnp.minimum(j, NB - 1)
        return pltpu.make_async_copy(
            xpad.at[src_v.at[pl.ds(jc * B, B)]], buf, gsem)

    def scat(j, buf, ssem):
        return pltpu.make_async_copy(buf, agg_s.at[dst2_v.at[j]], ssem)

    def scale_batch(buf, j):
        def sbody(e, _):
            nsp = plsc.load_gather(norm_v, [_splat(e) + j * B])
            buf[e, :] = buf[e, :] * nsp
            return 0
        lax.fori_loop(0, B, sbody, 0, unroll=2)

    gather(0, rows_a, gsa).start()
    gather(1, rows_b, gsb).start()

    def tri(jj, _):
        j0 = 3 * jj
        j1 = 3 * jj + 1
        j2 = 3 * jj + 2
        gather(j0, rows_a, gsa).wait()
        scale_batch(rows_a, j0)

        @pl.when(jj > 0)
        def _():
            scat(j2 - 3, rows_c, ssc).wait()
        gather(j2, rows_c, gsc).start()
        pltpu.async_copy(rows_a, agg_s.at[dst2_v.at[j0]], ssa, add=True)

        gather(j1, rows_b, gsb).wait()
        scale_batch(rows_b, j1)
        scat(j0, rows_a, ssa).wait()
        gather(j0 + 3, rows_a, gsa).start()
        pltpu.async_copy(rows_b, agg_s.at[dst2_v.at[j1]], ssb, add=True)

        gather(j2, rows_c, gsc).wait()
        scale_batch(rows_c, j2)
        scat(j1, rows_b, ssb).wait()
        gather(j1 + 3, rows_b, gsb).start()
        pltpu.async_copy(rows_c, agg_s.at[dst2_v.at[j2]], ssc, add=True)
        return 0
    lax.fori_loop(0, NB // 3, tri, 0)
    gather(NB - 1, rows_a, gsa).wait()
    gather(NB - 1, rows_b, gsb).wait()
    scat(NB - 1, rows_c, ssc).wait()

    plsc.subcore_barrier()
    pltpu.sync_copy(agg_s.at[sl], aggxp.at[c, sl])


def _tc_layer1(aggxp_ref, w1_ref, b1_ref, out_ref):
    a = aggxp_ref[0] + aggxp_ref[1]
    h = jnp.dot(a, w1_ref[:], preferred_element_type=f32) + b1_ref[:][None, :]
    h = jnp.maximum(h, 0.0)
    for cidx in range(NCH):
        out_ref[cidx] = h[:, cidx * CH:(cidx + 1) * CH]


def _sc_agg2(srcf, dstb, normf, ident4, z80, tp,
             src_v, dst2_v, norm_v, rows_a, rows_b, rows_c,
             acc_s, gsa, gsb, gsc, ssa, ssb, ssc):
    c = lax.axis_index("c")
    s = lax.axis_index("s")
    w = c * NS + s
    sl = pl.ds(s * SLICE, SLICE)
    pltpu.sync_copy(srcf.at[pl.ds(w * EPW, EPW)], src_v)
    pltpu.sync_copy(dstb.at[w], dst2_v)
    pltpu.sync_copy(normf.at[pl.ds(w * EPW, EPW)], norm_v)

    def scale_batch(buf, j):
        def sbody(e, _):
            nsp = plsc.load_gather(norm_v, [_splat(e) + j * B])
            for k in range(CH // 16):
                ds16 = pl.ds(k * 16, 16)
                buf[e, ds16] = buf[e, ds16] * nsp
            return 0
        lax.fori_loop(0, B, sbody, 0, unroll=2)

    for cidx in range(NCH):
        pltpu.sync_copy(z80.at[sl], acc_s.at[sl])
        plsc.subcore_barrier()

        def gather(j, buf, sem, cidx=cidx):
            jc = jnp.minimum(j, NB - 1)
            return pltpu.make_async_copy(
                ident4.at[cidx].at[src_v.at[pl.ds(jc * B, B)]], buf, sem)

        def scat(j, buf, sem):
            return pltpu.make_async_copy(buf, acc_s.at[dst2_v.at[j]], sem)

        gather(0, rows_a, gsa).start()
        gather(1, rows_b, gsb).start()

        def tri(jj, _):
            j0 = 3 * jj
            j1 = 3 * jj + 1
            j2 = 3 * jj + 2
            gather(j0, rows_a, gsa).wait()
            scale_batch(rows_a, j0)

            @pl.when(jj > 0)
            def _():
                scat(j2 - 3, rows_c, ssc).wait()
            gather(j2, rows_c, gsc).start()
            pltpu.async_copy(rows_a, acc_s.at[dst2_v.at[j0]], ssa, add=True)

            gather(j1, rows_b, gsb).wait()
            scale_batch(rows_b, j1)
            scat(j0, rows_a, ssa).wait()
            gather(j0 + 3, rows_a, gsa).start()
            pltpu.async_copy(rows_b, acc_s.at[dst2_v.at[j1]], ssb, add=True)

            gather(j2, rows_c, gsc).wait()
            scale_batch(rows_c, j2)
            scat(j1, rows_b, ssb).wait()
            gather(j1 + 3, rows_b, gsb).start()
            pltpu.async_copy(rows_c, acc_s.at[dst2_v.at[j2]], ssc, add=True)
            return 0
        lax.fori_loop(0, NB // 3, tri, 0)
        gather(NB - 1, rows_a, gsa).wait()
        gather(NB - 1, rows_b, gsb).wait()
        scat(NB - 1, rows_c, ssc).wait()
        plsc.subcore_barrier()
        pltpu.sync_copy(acc_s.at[sl], tp.at[c, cidx, sl])
        plsc.subcore_barrier()


def _tc_layer2(tp_ref, w2_ref, b2_ref, ident4_ref, we_ref, be_ref, out_ref):
    rows = tp_ref.shape[2]
    acc = jnp.zeros((rows, D), f32)
    for p in range(NC):
        for cidx in range(NCH):
            acc = acc + jnp.dot(tp_ref[p, cidx],
                                w2_ref[cidx * CH:(cidx + 1) * CH, :],
                                preferred_element_type=f32)
    h2 = jnp.maximum(acc + b2_ref[:][None, :], 0.0)
    o = jnp.dot(h2, we_ref[:D, :], preferred_element_type=f32)
    for cidx in range(NCH):
        o = o + jnp.dot(ident4_ref[cidx],
                        we_ref[D + cidx * CH:D + (cidx + 1) * CH, :],
                        preferred_element_type=f32)
    out_ref[:] = o + be_ref[:][None, :]


def kernel(x, edge_index, edge_weight, W1, b1, W2, b2, We, be):
    N, F = x.shape
    E = edge_weight.shape[0]

    src = edge_index[0].astype(i32)
    dst = edge_index[1].astype(i32)
    loop_idx = jnp.arange(N, dtype=i32)
    npad = EP - (E + N)
    pad_idx = (jnp.arange(npad, dtype=i32) * 37) % N
    srcf = jnp.concatenate([src, loop_idx, pad_idx])
    dstb = jnp.concatenate([dst, loop_idx, pad_idx]).reshape(NW, NB, B)
    ewf = jnp.concatenate([edge_weight.astype(f32), jnp.ones((N,), f32),
                           jnp.zeros((npad,), f32)])
    xpad = jnp.zeros((NP, 16), f32).at[:N, :F].set(x)
    w1p = jnp.zeros((16, D), f32).at[:F, :].set(W1)
    z16 = jnp.zeros((NP, 16), f32)
    z80 = jnp.zeros((NP, CH), f32)

    mesh = plsc.VectorSubcoreMesh(core_axis_name="c", subcore_axis_name="s")

    degp = pl.kernel(
        _sc_deg,
        out_type=jax.ShapeDtypeStruct((NC, NP, 16), f32),
        mesh=mesh,
        compiler_params=_SC_PARAMS,
        scratch_types=[
            pltpu.VMEM((NB, B), i32),
            pltpu.VMEM((EPW,), f32),
            pltpu.VMEM((B, 16), f32),
            pltpu.VMEM_SHARED((NP, 16), f32),
        ],
    )(dstb, ewf, z16)

    dinv = pl.pallas_call(
        _tc_dinv,
        out_shape=jax.ShapeDtypeStruct((NP,), f32),
    )(degp)

    normf, aggxp = pl.kernel(
        _sc_norm_aggx,
        out_type=(jax.ShapeDtypeStruct((EP,), f32),
                  jax.ShapeDtypeStruct((NC, NP, 16), f32)),
        mesh=mesh,
        compiler_params=_SC_PARAMS,
        scratch_types=[
            pltpu.VMEM((EPW,), i32),
            pltpu.VMEM((NB, B), i32),
            pltpu.VMEM((EPW,), f32),
            pltpu.VMEM((EPW,), f32),
            pltpu.VMEM((NP,), f32),
            pltpu.VMEM((B, 16), f32),
            pltpu.VMEM((B, 16), f32),
            pltpu.VMEM((B, 16), f32),
            pltpu.VMEM_SHARED((NP, 16), f32),
            pltpu.SemaphoreType.DMA,
            pltpu.SemaphoreType.DMA,
            pltpu.SemaphoreType.DMA,
            pltpu.SemaphoreType.DMA,
            pltpu.SemaphoreType.DMA,
            pltpu.SemaphoreType.DMA,
        ],
    )(srcf, dstb, ewf, dinv, xpad, z16)

    R = 512
    ident4 = pl.pallas_call(
        _tc_layer1,
        grid=(NP // R,),
        in_specs=[
            pl.BlockSpec((NC, R, 16), lambda i: (0, i, 0)),
            pl.BlockSpec((16, D), lambda i: (0, 0)),
            pl.BlockSpec((D,), lambda i: (0,)),
        ],
        out_specs=pl.BlockSpec((NCH, R, CH), lambda i: (0, i, 0)),
        out_shape=jax.ShapeDtypeStruct((NCH, NP, CH), f32),
    )(aggxp, w1p, b1)

    tp = pl.kernel(
        _sc_agg2,
        out_type=jax.ShapeDtypeStruct((NC, NCH, NP, CH), f32),
        mesh=mesh,
        compiler_params=_SC_PARAMS,
        scratch_types=[
            pltpu.VMEM((EPW,), i32),
            pltpu.VMEM((NB, B), i32),
            pltpu.VMEM((EPW,), f32),
            pltpu.VMEM((B, CH), f32),
            pltpu.VMEM((B, CH), f32),
            pltpu.VMEM((B, CH), f32),
            pltpu.VMEM_SHARED((NP, CH), f32),
            pltpu.SemaphoreType.DMA,
            pltpu.SemaphoreType.DMA,
            pltpu.SemaphoreType.DMA,
            pltpu.SemaphoreType.DMA,
            pltpu.SemaphoreType.DMA,
            pltpu.SemaphoreType.DMA,
        ],
    )(srcf, dstb, normf, ident4, z80)

    outp = pl.pallas_call(
        _tc_layer2,
        grid=(NP // R,),
        in_specs=[
            pl.BlockSpec((NC, NCH, R, CH), lambda i: (0, 0, i, 0)),
            pl.BlockSpec((D, D), lambda i: (0, 0)),
            pl.BlockSpec((D,), lambda i: (0,)),
            pl.BlockSpec((NCH, R, CH), lambda i: (0, i, 0)),
            pl.BlockSpec((2 * D, 80), lambda i: (0, 0)),
            pl.BlockSpec((80,), lambda i: (0,)),
        ],
        out_specs=pl.BlockSpec((R, 80), lambda i: (i, 0)),
        out_shape=jax.ShapeDtypeStruct((NP, 80), f32),
    )(tp, W2, b2, ident4, We, be)

    return outp[:N]
